# Initial kernel scaffold; baseline (speedup 1.0000x reference)
import jax
import jax.numpy as jnp
from jax.experimental import pallas as pl


def kernel(input, mu, rho, eps):
    raise NotImplementedError("write your pallas kernel here")


# speedup vs baseline: 2.7741x; 2.7741x over previous
import functools
import math

import jax
import jax.numpy as jnp
from jax import lax
from jax.experimental import pallas as pl
from jax.experimental.pallas import tpu as pltpu
from jax.experimental.pallas import tpu_sc as plsc

N_EMB = 100000
EMB_DIM = 64
SIGMA1 = 1.0
SIGMA2 = 0.1
PI_MIX = 0.5

_ROWS = (N_EMB * EMB_DIM) // 128
_BLK = 2000
_GRID = _ROWS // _BLK

_NC, _NS = 2, 16
_NW = _NC * _NS
_CH = 128
_B_TOTAL = 4096 * 50
_J = _B_TOTAL // (_NW * _CH)


def _dense_body(mu_ref, rho_ref, eps_ref, w_ref, sums_ref, acc_ref):
    step = pl.program_id(0)

    @pl.when(step == 0)
    def _init():
        acc_ref[0] = jnp.float32(0)
        acc_ref[1] = jnp.float32(0)
        acc_ref[2] = jnp.float32(0)

    rho = rho_ref[...]
    eps = eps_ref[...]
    sigma = jax.nn.softplus(rho) + 1e-5
    w = mu_ref[...] + eps * sigma
    w_ref[...] = w
    acc_ref[0] = acc_ref[0] + jnp.sum(jnp.log(sigma))
    acc_ref[1] = acc_ref[1] + jnp.sum(eps * eps)
    acc_ref[2] = acc_ref[2] + jnp.sum(w * w)

    @pl.when(step == _GRID - 1)
    def _fin():
        sums_ref[0] = acc_ref[0]
        sums_ref[1] = acc_ref[1]
        sums_ref[2] = acc_ref[2]


_dense = pl.pallas_call(
    _dense_body,
    grid=(_GRID,),
    in_specs=[pl.BlockSpec((_BLK, 128), lambda i: (i, 0))] * 3,
    out_specs=[
        pl.BlockSpec((_BLK, 128), lambda i: (i, 0)),
        pl.BlockSpec(memory_space=pltpu.SMEM),
    ],
    out_shape=[
        jax.ShapeDtypeStruct((_ROWS, 128), jnp.float32),
        jax.ShapeDtypeStruct((3,), jnp.float32),
    ],
    scratch_shapes=[pltpu.SMEM((3,), jnp.float32)],
)


@functools.cache
def _make_sc_gather():
    mesh = plsc.VectorSubcoreMesh(
        core_axis_name="c", subcore_axis_name="s", num_cores=_NC, num_subcores=_NS
    )

    @functools.partial(
        pl.kernel,
        out_type=jax.ShapeDtypeStruct((_B_TOTAL, EMB_DIM), jnp.float32),
        mesh=mesh,
        scratch_types=[
            pltpu.VMEM((_J, _CH), jnp.int32),
            pltpu.VMEM((_CH, EMB_DIM), jnp.float32),
            pltpu.SemaphoreType.DMA,
        ],
        compiler_params=pltpu.CompilerParams(use_tc_tiling_on_sc=False),
    )
    def _sc_gather(table_hbm, idx_hbm, out_hbm, idx_v, rows_v, sem):
        wid = lax.axis_index("s") * _NC + lax.axis_index("c")
        base = wid * (_J * _CH)
        pltpu.sync_copy(idx_hbm.at[wid], idx_v)

        def body(j, carry):
            pltpu.async_copy(table_hbm.at[idx_v.at[j]], rows_v, sem).wait()
            pltpu.sync_copy(rows_v, out_hbm.at[pl.ds(base + j * _CH, _CH)])
            return carry

        lax.fori_loop(0, _J, body, 0)

    return _sc_gather


def kernel(input, mu, rho, eps):
    mu2 = mu.reshape(_ROWS, 128)
    rho2 = rho.reshape(_ROWS, 128)
    eps2 = eps.reshape(_ROWS, 128)
    w2, sums = _dense(mu2, rho2, eps2)
    weights = w2.reshape(N_EMB, EMB_DIM)

    idx3 = input.astype(jnp.int32).reshape(_NW, _J, _CH)
    flat = _make_sc_gather()(weights, idx3)
    after_embed = flat.reshape(input.shape[0], input.shape[1], EMB_DIM)

    s_logsig, s_eps2, s_w2 = sums[0], sums[1], sums[2]
    n = float(N_EMB * EMB_DIM)
    c = 0.5 * math.log(2.0 * math.pi)
    log_posterior = -s_logsig - n * c - 0.5 * s_eps2
    mix1 = (
        -n * math.log(SIGMA1) - n * c - 0.5 * s_w2 / (SIGMA1 * SIGMA1)
        + math.log(PI_MIX)
    )
    mix2 = (
        -n * math.log(SIGMA2) - n * c - 0.5 * s_w2 / (SIGMA2 * SIGMA2)
        + math.log(1.0 - PI_MIX)
    )
    log_prior = jnp.logaddexp(mix1, mix2)
    kl = log_posterior - log_prior
    return (after_embed, kl)
